# baseline (speedup 1.0000x reference)
import functools

import jax
import jax.numpy as jnp
from jax import lax
from jax.experimental import pallas as pl
from jax.experimental.pallas import tpu as pltpu
from jax.experimental.pallas import tpu_sc as plsc

_NE = 8192
_D = 256
_N = 8192
_BN = 512
_NBLK = _N // _BN
_COMMIT_BETA = 0.25

_SC_NC = 2
_SC_NS = 16
_SC_NW = _SC_NC * _SC_NS
_B_PER_W = _N // _SC_NW


def _w_norms_body(w_ref, w2_ref, kidx_ref):
    w = w_ref[...]
    w2_ref[...] = jnp.sum(w * w, axis=1, keepdims=True)
    kidx_ref[...] = lax.broadcasted_iota(
        jnp.int32, (1, _NE), 1).astype(jnp.float32)


def _vq_body(z_ref, w_ref, w2_ref, kidx_ref, ids_ref, dsum_ref, d_scr):
    dprev = d_scr[...]
    m = jnp.min(dprev, axis=1, keepdims=True)
    first = jnp.min(jnp.where(dprev <= m, kidx_ref[...], jnp.float32(_NE)),
                    axis=1, keepdims=True)
    ids_ref[...] = first.astype(jnp.int32)
    dsum_ref[0, 0, 0] = jnp.sum(m)

    z = z_ref[...]
    z2 = jnp.sum(z * z, axis=1, keepdims=True)
    b2 = lax.dot_general(z + z, w_ref[...], (((1,), (1,)), ((), ())),
                         preferred_element_type=jnp.float32)
    d_scr[...] = (z2 - b2) + w2_ref[...]


def _make_sc_gather(nrows):
    b_per_w = nrows // _SC_NW
    mesh = plsc.VectorSubcoreMesh(core_axis_name="c", subcore_axis_name="s")

    @functools.partial(
        pl.kernel,
        mesh=mesh,
        out_type=jax.ShapeDtypeStruct((nrows, _D), jnp.float32),
        scratch_types=[
            pltpu.VMEM((b_per_w,), jnp.int32),
            pltpu.VMEM((b_per_w, _D), jnp.float32),
            pltpu.SemaphoreType.DMA,
        ],
    )
    def _sc_gather(table_hbm, idx_hbm, out_hbm, idx_v, rows_v, sem):
        wid = lax.axis_index("s") * _SC_NC + lax.axis_index("c")
        base = wid * b_per_w
        pltpu.sync_copy(idx_hbm.at[pl.ds(base, b_per_w)], idx_v)
        pltpu.async_copy(table_hbm.at[idx_v], rows_v, sem).wait()
        pltpu.sync_copy(rows_v, out_hbm.at[pl.ds(base, b_per_w)])

    return _sc_gather


def kernel(input, W):
    w2, kidx = pl.pallas_call(
        _w_norms_body,
        out_shape=[
            jax.ShapeDtypeStruct((_NE, 1), jnp.float32),
            jax.ShapeDtypeStruct((1, _NE), jnp.float32),
        ],
    )(W)
    w2t = w2.reshape(1, _NE)

    nchunk = 1
    rows = _N // nchunk
    nblk = rows // _BN
    sc_gather = _make_sc_gather(rows)

    q_parts = []
    dsum_parts = []
    for c in range(nchunk):
        ids, dsum = pl.pallas_call(
            _vq_body,
            grid=(nblk + 1,),
            in_specs=[
                pl.BlockSpec((_BN, _D),
                             lambda i: (jnp.minimum(i, nblk - 1), 0)),
                pl.BlockSpec((_NE, _D), lambda i: (0, 0)),
                pl.BlockSpec((1, _NE), lambda i: (0, 0)),
                pl.BlockSpec((1, _NE), lambda i: (0, 0)),
            ],
            out_specs=[
                pl.BlockSpec((_BN, 1),
                             lambda i: (jnp.maximum(i - 1, 0), 0)),
                pl.BlockSpec((1, 1, 1),
                             lambda i: (jnp.maximum(i - 1, 0), 0, 0),
                             memory_space=pltpu.SMEM),
            ],
            out_shape=[
                jax.ShapeDtypeStruct((rows, 1), jnp.int32),
                jax.ShapeDtypeStruct((nblk, 1, 1), jnp.float32),
            ],
            scratch_shapes=[pltpu.VMEM((_BN, _NE), jnp.float32)],
            compiler_params=pltpu.CompilerParams(
                dimension_semantics=("arbitrary",),
            ),
        )(input[c * rows:(c + 1) * rows], W, w2t, kidx)
        q_parts.append(sc_gather(W, ids.reshape(rows)))
        dsum_parts.append(dsum)

    quantized = jnp.concatenate(q_parts, axis=0)
    loss_mean = (jnp.sum(jnp.stack(dsum_parts))
                 / jnp.float32(_N * _D))
    vq_loss = loss_mean * _COMMIT_BETA + loss_mean
    return (quantized, vq_loss)


# speedup vs baseline: 1.0452x; 1.0452x over previous
import functools

import jax
import jax.numpy as jnp
from jax import lax
from jax.experimental import pallas as pl
from jax.experimental.pallas import tpu as pltpu
from jax.experimental.pallas import tpu_sc as plsc

_NE = 8192
_D = 256
_N = 8192
_BN = 512
_NBLK = _N // _BN
_COMMIT_BETA = 0.25

_SC_NC = 2
_SC_NS = 16
_SC_NW = _SC_NC * _SC_NS
_B_PER_W = _N // _SC_NW


def _w_norms_body(w_ref, w2_ref, kidx_ref):
    w = w_ref[...]
    w2_ref[...] = jnp.sum(w * w, axis=1, keepdims=True)
    kidx_ref[...] = lax.broadcasted_iota(
        jnp.int32, (1, _NE), 1).astype(jnp.float32)


def _vq_body(z_ref, w_ref, w2_ref, kidx_ref, ids_ref, m_ref):
    z = z_ref[...]
    z2 = jnp.sum(z * z, axis=1, keepdims=True)
    b2 = lax.dot_general(z + z, w_ref[...], (((1,), (1,)), ((), ())),
                         preferred_element_type=jnp.float32)
    d = (z2 - b2) + w2_ref[...]
    m = jnp.min(d, axis=1, keepdims=True)
    first = jnp.min(jnp.where(d <= m, kidx_ref[...], jnp.float32(_NE)),
                    axis=1, keepdims=True)
    ids_ref[...] = first.astype(jnp.int32)
    m_ref[...] = m


def _make_sc_gather(nrows):
    b_per_w = nrows // _SC_NW
    mesh = plsc.VectorSubcoreMesh(core_axis_name="c", subcore_axis_name="s")

    @functools.partial(
        pl.kernel,
        mesh=mesh,
        out_type=jax.ShapeDtypeStruct((nrows, _D), jnp.float32),
        scratch_types=[
            pltpu.VMEM((b_per_w,), jnp.int32),
            pltpu.VMEM((b_per_w, _D), jnp.float32),
            pltpu.SemaphoreType.DMA,
        ],
    )
    def _sc_gather(table_hbm, idx_hbm, out_hbm, idx_v, rows_v, sem):
        wid = lax.axis_index("s") * _SC_NC + lax.axis_index("c")
        base = wid * b_per_w
        pltpu.sync_copy(idx_hbm.at[pl.ds(base, b_per_w)], idx_v)
        pltpu.async_copy(table_hbm.at[idx_v], rows_v, sem).wait()
        pltpu.sync_copy(rows_v, out_hbm.at[pl.ds(base, b_per_w)])

    return _sc_gather


def kernel(input, W):
    w2, kidx = pl.pallas_call(
        _w_norms_body,
        out_shape=[
            jax.ShapeDtypeStruct((_NE, 1), jnp.float32),
            jax.ShapeDtypeStruct((1, _NE), jnp.float32),
        ],
    )(W)
    w2t = w2.reshape(1, _NE)

    nchunk = 1
    rows = _N // nchunk
    nblk = rows // _BN
    sc_gather = _make_sc_gather(rows)

    q_parts = []
    dsum_parts = []
    for c in range(nchunk):
        ids, mmin = pl.pallas_call(
            _vq_body,
            grid=(nblk,),
            in_specs=[
                pl.BlockSpec((_BN, _D), lambda i: (i, 0)),
                pl.BlockSpec((_NE, _D), lambda i: (0, 0)),
                pl.BlockSpec((1, _NE), lambda i: (0, 0)),
                pl.BlockSpec((1, _NE), lambda i: (0, 0)),
            ],
            out_specs=[
                pl.BlockSpec((_BN, 1), lambda i: (i, 0)),
                pl.BlockSpec((_BN, 1), lambda i: (i, 0)),
            ],
            out_shape=[
                jax.ShapeDtypeStruct((rows, 1), jnp.int32),
                jax.ShapeDtypeStruct((rows, 1), jnp.float32),
            ],
            compiler_params=pltpu.CompilerParams(
                dimension_semantics=("arbitrary",),
            ),
        )(input[c * rows:(c + 1) * rows], W, w2t, kidx)
        q_parts.append(sc_gather(W, ids.reshape(rows)))
        dsum_parts.append(mmin)

    quantized = jnp.concatenate(q_parts, axis=0)
    loss_mean = (jnp.sum(jnp.stack(dsum_parts))
                 / jnp.float32(_N * _D))
    vq_loss = loss_mean * _COMMIT_BETA + loss_mean
    return (quantized, vq_loss)
